# Initial kernel scaffold; baseline (speedup 1.0000x reference)
import jax
import jax.numpy as jnp
from jax.experimental import pallas as pl


def kernel(x, edge_index, W1, b1, W2, b2):
    raise NotImplementedError("write your pallas kernel here")


# speedup vs baseline: 5.8715x; 5.8715x over previous
import functools

import jax
import jax.numpy as jnp
from jax import lax
from jax.experimental import pallas as pl
from jax.experimental.pallas import tpu as pltpu
from jax.experimental.pallas import tpu_sc as plsc

N = 10000
E = 160000
IN_DIM = 256
HIDDEN = 512
OUT_DIM = 256

LANE = 128
NC, NS = 2, 16
EB = 128
NACC = 10112
ROWS_Z = NACC // NS
ROWS_O = N // NS

NB = 80
E_PAD = NS * NB * EB
NBD = 40
E_PADD = NC * NS * NBD * EB

BM = 2000


def _deg_body(dst2_hbm, zdeg_hbm, out_hbm, dacc, idx_v, ones_v, dbuf):
    cid = lax.axis_index("c")
    sid = lax.axis_index("s")
    wid = cid * NS + sid
    pltpu.sync_copy(zdeg_hbm.at[pl.ds(sid * ROWS_Z, ROWS_Z)], dbuf)
    pltpu.sync_copy(dbuf, dacc.at[pl.ds(sid * ROWS_Z, ROWS_Z)])
    for k in range(EB // 16):
        ones_v[pl.ds(16 * k, 16)] = jnp.ones((16,), jnp.float32)
    pltpu.sync_copy(dst2_hbm.at[pl.ds(wid * NBD, NBD)], idx_v)
    plsc.subcore_barrier()

    def step(j, carry):
        pltpu.sync_copy(ones_v, dacc.at[idx_v.at[j]], add=True)
        return carry

    lax.fori_loop(0, NBD, step, 0)
    plsc.subcore_barrier()
    pltpu.sync_copy(dacc.at[pl.ds(sid * ROWS_Z, ROWS_Z)], dbuf)
    pltpu.sync_copy(dbuf, out_hbm.at[pl.ds(cid * NACC + sid * ROWS_Z, ROWS_Z)])


_deg_kernel = functools.partial(
    pl.kernel,
    out_type=jax.ShapeDtypeStruct((NC * NACC,), jnp.float32),
    mesh=plsc.VectorSubcoreMesh(core_axis_name="c", subcore_axis_name="s", num_cores=NC, num_subcores=NS),
    scratch_types=[
        pltpu.VMEM_SHARED((NACC,), jnp.float32),
        pltpu.VMEM((NBD, EB), jnp.int32),
        pltpu.VMEM((EB,), jnp.float32),
        pltpu.VMEM((ROWS_Z,), jnp.float32),
    ],
)(_deg_body)


def _make_agg(C):
    cpc = C // NC

    pieces = [(o, min(EB, ROWS_Z - o)) for o in range(0, ROWS_Z, EB)]

    def body(g2d_hbm, src3_hbm, dst2_hbm, z2d_hbm, out_hbm,
             acc, sidx, didx, rows):
        cid = lax.axis_index("c")
        sid = lax.axis_index("s")
        for i in range(cpc):
            ck = cid + NC * i
            for off, sz in pieces:
                pltpu.sync_copy(z2d_hbm.at[pl.ds(0, sz)],
                                rows.at[pl.ds(0, sz)])
                pltpu.sync_copy(rows.at[pl.ds(0, sz)],
                                acc.at[pl.ds(sid * ROWS_Z + off, sz)])
            pltpu.sync_copy(src3_hbm.at[ck, pl.ds(sid * NB, NB)], sidx)
            pltpu.sync_copy(dst2_hbm.at[pl.ds(sid * NB, NB)], didx)
            plsc.subcore_barrier()

            def step(j, carry):
                pltpu.sync_copy(g2d_hbm.at[sidx.at[j]], rows)
                pltpu.sync_copy(rows, acc.at[didx.at[j]], add=True)
                return carry

            lax.fori_loop(0, NB, step, 0)
            plsc.subcore_barrier()
            for off, sz in pieces:
                pltpu.sync_copy(acc.at[pl.ds(sid * ROWS_Z + off, sz)],
                                rows.at[pl.ds(0, sz)])
                pltpu.sync_copy(rows.at[pl.ds(0, sz)],
                                out_hbm.at[ck, pl.ds(sid * ROWS_Z + off, sz)])
            plsc.subcore_barrier()

    return functools.partial(
        pl.kernel,
        out_type=jax.ShapeDtypeStruct((C, NACC, LANE), jnp.float32),
        mesh=plsc.VectorSubcoreMesh(core_axis_name="c", subcore_axis_name="s", num_cores=NC, num_subcores=NS),
        scratch_types=[
            pltpu.VMEM_SHARED((NACC, LANE), jnp.float32),
            pltpu.VMEM((NB, EB), jnp.int32),
            pltpu.VMEM((NB, EB), jnp.int32),
            pltpu.VMEM((EB, LANE), jnp.float32),
        ],
    )(body)


_agg4 = _make_agg(4)
_agg2 = _make_agg(2)


def _dis_of(degt_ref):
    return lax.rsqrt(degt_ref[:, 0:1] + degt_ref[:, 1:2] + 1.0)


def _tc1_body(degt_ref, x_ref, w_ref, o_ref):
    dis = _dis_of(degt_ref)
    h = jnp.dot(x_ref[...], w_ref[...], preferred_element_type=jnp.float32)
    o_ref[0] = dis * h


def _tc2_body(degt_ref, agg_ref, g_ref, b_ref, w_ref, o_ref):
    dis = _dis_of(degt_ref)
    acc = jnp.zeros((BM, LANE), jnp.float32)
    for c in range(HIDDEN // LANE):
        h = dis * (agg_ref[c] + g_ref[c]) + b_ref[c]
        h = jnp.maximum(h, 0.0)
        acc = acc + jnp.dot(h, w_ref[c], preferred_element_type=jnp.float32)
    o_ref[0] = dis * acc


def _tc3_body(degt_ref, agg_ref, g_ref, b_ref, o_ref):
    dis = _dis_of(degt_ref)
    left = dis * (agg_ref[0] + g_ref[0]) + b_ref[0]
    right = dis * (agg_ref[1] + g_ref[1]) + b_ref[1]
    o_ref[...] = jnp.concatenate([left, right], axis=1)


_C1 = HIDDEN // LANE
_C2 = OUT_DIM // LANE
_GM = N // BM

_tc1 = pl.pallas_call(
    _tc1_body,
    grid=(_GM, _C1),
    in_specs=[
        pl.BlockSpec((BM, 2), lambda i, j: (i, 0)),
        pl.BlockSpec((BM, IN_DIM), lambda i, j: (i, 0)),
        pl.BlockSpec((IN_DIM, LANE), lambda i, j: (0, j)),
    ],
    out_specs=pl.BlockSpec((1, BM, LANE), lambda i, j: (j, i, 0)),
    out_shape=jax.ShapeDtypeStruct((_C1, N, LANE), jnp.float32),
)

_tc2 = pl.pallas_call(
    _tc2_body,
    grid=(_GM, _C2),
    in_specs=[
        pl.BlockSpec((BM, 2), lambda i, j: (i, 0)),
        pl.BlockSpec((_C1, BM, LANE), lambda i, j: (0, i, 0)),
        pl.BlockSpec((_C1, BM, LANE), lambda i, j: (0, i, 0)),
        pl.BlockSpec((_C1, 1, LANE), lambda i, j: (0, 0, 0)),
        pl.BlockSpec((_C1, LANE, LANE), lambda i, j: (0, 0, j)),
    ],
    out_specs=pl.BlockSpec((1, BM, LANE), lambda i, j: (j, i, 0)),
    out_shape=jax.ShapeDtypeStruct((_C2, N, LANE), jnp.float32),
)

_tc3 = pl.pallas_call(
    _tc3_body,
    grid=(_GM,),
    in_specs=[
        pl.BlockSpec((BM, 2), lambda i: (i, 0)),
        pl.BlockSpec((_C2, BM, LANE), lambda i: (0, i, 0)),
        pl.BlockSpec((_C2, BM, LANE), lambda i: (0, i, 0)),
        pl.BlockSpec((_C2, 1, LANE), lambda i: (0, 0, 0)),
    ],
    out_specs=pl.BlockSpec((BM, OUT_DIM), lambda i: (i, 0)),
    out_shape=jax.ShapeDtypeStruct((N, OUT_DIM), jnp.float32),
)


def kernel(x, edge_index, W1, b1, W2, b2):
    src = edge_index[0].astype(jnp.int32)
    dst = edge_index[1].astype(jnp.int32)

    dstd = jnp.concatenate(
        [dst, jnp.full((E_PADD - E,), N, jnp.int32)]).reshape(-1, EB)
    zdeg = jnp.zeros((NACC,), jnp.float32)
    degp = _deg_kernel(dstd, zdeg).reshape(NC, NACC)
    degt = degp[:, :N].T

    srcp = jnp.concatenate([src, jnp.zeros((E_PAD - E,), jnp.int32)])
    dst2 = jnp.concatenate(
        [dst, jnp.full((E_PAD - E,), N, jnp.int32)]).reshape(-1, EB)
    off1 = (jnp.arange(_C1, dtype=jnp.int32) * N)[:, None]
    off2 = (jnp.arange(_C2, dtype=jnp.int32) * N)[:, None]
    src31 = (srcp[None, :] + off1).reshape(_C1, -1, EB)
    src32 = (srcp[None, :] + off2).reshape(_C2, -1, EB)
    z2d = jnp.zeros((EB, LANE), jnp.float32)

    g1 = _tc1(degt, x, W1)
    agg1 = _agg4(g1.reshape(_C1 * N, LANE), src31, dst2, z2d)
    g2 = _tc2(degt, agg1, g1, b1.reshape(_C1, 1, LANE),
              W2.reshape(_C1, LANE, OUT_DIM))
    agg2 = _agg2(g2.reshape(_C2 * N, LANE), src32, dst2, z2d)
    out = _tc3(degt, agg2, g2, b2.reshape(_C2, 1, LANE))
    return out
